# baseline (speedup 1.0000x reference)
import functools

import jax
import jax.numpy as jnp
from jax import lax
from jax.experimental import pallas as pl
from jax.experimental.pallas import tpu as pltpu
from jax.experimental.pallas import tpu_sc as plsc

Y_LOW = 0.0
Y_HIGH = 10.5
L = 16


def _make_sc_kernel(B, NC, NS):
    NW = NC * NS
    bpw = B // NW
    ngroups = bpw // L
    mesh = plsc.VectorSubcoreMesh(core_axis_name="c", subcore_axis_name="s")

    @functools.partial(
        pl.kernel,
        mesh=mesh,
        compiler_params=pltpu.CompilerParams(use_tc_tiling_on_sc=False),
        out_type=jax.ShapeDtypeStruct((B,), jnp.float32),
        scratch_types=[
            pltpu.VMEM((bpw,), jnp.int32),
            pltpu.VMEM((bpw,), jnp.int32),
            pltpu.VMEM((L, bpw), jnp.float32),
            pltpu.VMEM((L, bpw), jnp.float32),
            pltpu.VMEM((bpw,), jnp.float32),
            pltpu.VMEM((bpw,), jnp.float32),
            pltpu.VMEM((bpw,), jnp.float32),
            pltpu.SemaphoreType.DMA,
        ],
    )
    def sc_kernel(uid_hbm, mid_hbm, uft_hbm, ub_hbm, mft_hbm, mb_hbm, out_hbm,
                  uid_v, mid_v, ufc, mfc, ubr, mbr, outv, sem):
        sid = lax.axis_index("s")
        wid = sid * NC + lax.axis_index("c")
        base = wid * bpw
        pltpu.sync_copy(uid_hbm.at[pl.ds(base, bpw)], uid_v)
        pltpu.sync_copy(mid_hbm.at[pl.ds(base, bpw)], mid_v)
        copies = []
        for k in range(L):
            copies.append(
                pltpu.async_copy(uft_hbm.at[k].at[uid_v], ufc.at[k], sem))
            copies.append(
                pltpu.async_copy(mft_hbm.at[k].at[mid_v], mfc.at[k], sem))
        copies.append(pltpu.async_copy(ub_hbm.at[uid_v], ubr, sem))
        copies.append(pltpu.async_copy(mb_hbm.at[mid_v], mbr, sem))
        for c in copies:
            c.wait()

        for g in range(ngroups):
            sl = pl.ds(g * L, L)
            acc = ubr[sl] + mbr[sl]
            for k in range(L):
                acc = acc + ufc[k, sl] * mfc[k, sl]
            outv[sl] = Y_HIGH / (1.0 + jnp.exp(-acc)) + Y_LOW

        pltpu.sync_copy(outv, out_hbm.at[pl.ds(base, bpw)])

    return sc_kernel


def kernel(x, user_factors, user_bias, movie_factors, movie_bias):
    B = x.shape[0]
    info = plsc.get_sparse_core_info()
    sc_kernel = _make_sc_kernel(B, info.num_cores, info.num_subcores)
    uid = x[:, 0]
    mid = x[:, 1]
    return sc_kernel(uid, mid, user_factors.T, user_bias,
                     movie_factors.T, movie_bias)
